# baseline (speedup 1.0000x reference)
import functools

import jax
import jax.numpy as jnp
from jax import lax
from jax.experimental import pallas as pl
from jax.experimental.pallas import tpu as pltpu
from jax.experimental.pallas import tpu_sc as plsc

_NUM_CORES = 2
_NUM_SUBCORES = 16
_NUM_WORKERS = _NUM_CORES * _NUM_SUBCORES
_BBLK = 512
_L = 16


@functools.lru_cache(maxsize=None)
def _gather_call(NB, NS, V, D):
    assert D == 32 and NB % _BBLK == 0
    n_units = NS * (NB // _BBLK)
    units_per_w = n_units // _NUM_WORKERS
    assert units_per_w * _NUM_WORKERS == n_units
    blk_per_s = NB // _BBLK
    DT, DI = D // 8, 8
    BT = _BBLK // 128
    mesh = plsc.VectorSubcoreMesh(core_axis_name="c", subcore_axis_name="s")

    @functools.partial(
        pl.kernel,
        out_type=jax.ShapeDtypeStruct((NS, DT, NB // 128, DI, 128), jnp.float32),
        mesh=mesh,
        scratch_types=[
            pltpu.VMEM((_BBLK,), jnp.int32),
            pltpu.VMEM((_BBLK,), jnp.int32),
            pltpu.VMEM((_BBLK, D), jnp.float32),
            pltpu.VMEM((_BBLK, D), jnp.float32),
            pltpu.VMEM((DT, BT, DI, 129), jnp.float32),
            pltpu.VMEM((DT, BT, DI, 129), jnp.float32),
            pltpu.SemaphoreType.DMA,
            pltpu.SemaphoreType.DMA,
        ],
        compiler_params=pltpu.CompilerParams(
            use_tc_tiling_on_sc=False, needs_layout_passes=False
        ),
    )
    def body(
        idx_hbm, table_hbm, out_hbm, idx0, idx1, rows0, rows1, t0, t1, sem0, sem1
    ):
        wid = lax.axis_index("s") * _NUM_CORES + lax.axis_index("c")
        lanes = lax.iota(jnp.int32, 16)
        dtv = [2 * h + lanes // 8 for h in (0, 1)]
        div = lanes % 8
        bufs = ((idx0, rows0, t0, sem0), (idx1, rows1, t1, sem1))

        def start(b, u):
            idx_v, rows_v, _, sem = bufs[b]
            s = u // blk_per_s
            blk = u % blk_per_s
            pltpu.sync_copy(
                idx_hbm.at[pl.ds(s * NB + blk * _BBLK, _BBLK)], idx_v
            )
            pltpu.async_copy(table_hbm.at[idx_v], rows_v, sem)

        def finish(b, u):
            idx_v, rows_v, t_v, sem = bufs[b]
            s = u // blk_per_s
            blk = u % blk_per_s
            pltpu.make_async_copy(table_hbm.at[idx_v], rows_v, sem).wait()

            @plsc.parallel_loop(0, _BBLK, step=8, unroll=4)
            def _rows(r0):
                for rr in range(8):
                    r = r0 + rr
                    btv = jnp.full((16,), r // 128, dtype=jnp.int32)
                    biv = jnp.full((16,), r % 128, dtype=jnp.int32)
                    for h in (0, 1):
                        vec = rows_v[r, pl.ds(16 * h, 16)]
                        plsc.store_scatter(t_v, [dtv[h], btv, div, biv], vec)

            for dt in range(DT):
                pltpu.sync_copy(
                    t_v.at[dt, :, :, pl.ds(0, 128)],
                    out_hbm.at[s, dt, pl.ds(blk * BT, BT)],
                )

        u0 = wid * units_per_w
        start(0, u0)

        @pl.loop(0, units_per_w // 2)
        def _pair(p):
            k = u0 + 2 * p
            start(1, k + 1)
            finish(0, k)

            @pl.when(2 * p + 2 < units_per_w)
            def _prefetch():
                start(0, k + 2)

            finish(1, k + 1)

    return body


def kernel(input, weight):
    NB, NS = input.shape
    V, D = weight.shape
    flat_idx = input.T.reshape(NB * NS)
    a5 = _gather_call(NB, NS, V, D)(flat_idx, weight)
    return a5.transpose(2, 4, 0, 1, 3).reshape(NB, NS, D)


# speedup vs baseline: 1.0607x; 1.0607x over previous
import functools

import jax
import jax.numpy as jnp
from jax import lax
from jax.experimental import pallas as pl
from jax.experimental.pallas import tpu as pltpu
from jax.experimental.pallas import tpu_sc as plsc

_NUM_CORES = 2
_NUM_SUBCORES = 16
_NUM_WORKERS = _NUM_CORES * _NUM_SUBCORES
_BBLK = 1024
_L = 16


@functools.lru_cache(maxsize=None)
def _gather_call(NB, NS, V, D):
    assert D == 32 and NB % _BBLK == 0
    n_units = NS * (NB // _BBLK)
    units_per_w = n_units // _NUM_WORKERS
    assert units_per_w * _NUM_WORKERS == n_units
    blk_per_s = NB // _BBLK
    DT, DI = D // 8, 8
    BT = _BBLK // 128
    mesh = plsc.VectorSubcoreMesh(core_axis_name="c", subcore_axis_name="s")

    @functools.partial(
        pl.kernel,
        out_type=jax.ShapeDtypeStruct((NS, DT, NB // 128, DI, 128), jnp.float32),
        mesh=mesh,
        scratch_types=[
            pltpu.VMEM((_BBLK,), jnp.int32),
            pltpu.VMEM((_BBLK,), jnp.int32),
            pltpu.VMEM((_BBLK, D), jnp.float32),
            pltpu.VMEM((_BBLK, D), jnp.float32),
            pltpu.VMEM((DT, BT, DI, 129), jnp.float32),
            pltpu.SemaphoreType.DMA,
            pltpu.SemaphoreType.DMA,
        ],
        compiler_params=pltpu.CompilerParams(
            use_tc_tiling_on_sc=False, needs_layout_passes=False
        ),
    )
    def body(
        idx_hbm, table_hbm, out_hbm, idx0, idx1, rows0, rows1, t_v, sem0, sem1
    ):
        wid = lax.axis_index("s") * _NUM_CORES + lax.axis_index("c")
        lanes = lax.iota(jnp.int32, 16)
        dtv = [2 * h + lanes // 8 for h in (0, 1)]
        div = lanes % 8
        bufs = ((idx0, rows0, t_v, sem0), (idx1, rows1, t_v, sem1))

        def start(b, u):
            idx_v, rows_v, _, sem = bufs[b]
            s = u // blk_per_s
            blk = u % blk_per_s
            pltpu.sync_copy(
                idx_hbm.at[pl.ds(s * NB + blk * _BBLK, _BBLK)], idx_v
            )
            pltpu.async_copy(table_hbm.at[idx_v], rows_v, sem)

        def finish(b, u):
            idx_v, rows_v, t_v, sem = bufs[b]
            s = u // blk_per_s
            blk = u % blk_per_s
            pltpu.make_async_copy(table_hbm.at[idx_v], rows_v, sem).wait()

            @plsc.parallel_loop(0, _BBLK, step=8, unroll=4)
            def _rows(r0):
                for rr in range(8):
                    r = r0 + rr
                    btv = jnp.full((16,), r // 128, dtype=jnp.int32)
                    biv = jnp.full((16,), r % 128, dtype=jnp.int32)
                    for h in (0, 1):
                        vec = rows_v[r, pl.ds(16 * h, 16)]
                        plsc.store_scatter(t_v, [dtv[h], btv, div, biv], vec)

            for dt in range(DT):
                pltpu.sync_copy(
                    t_v.at[dt, :, :, pl.ds(0, 128)],
                    out_hbm.at[s, dt, pl.ds(blk * BT, BT)],
                )

        u0 = wid * units_per_w
        start(0, u0)

        @pl.loop(0, units_per_w // 2)
        def _pair(p):
            k = u0 + 2 * p
            start(1, k + 1)
            finish(0, k)

            @pl.when(2 * p + 2 < units_per_w)
            def _prefetch():
                start(0, k + 2)

            finish(1, k + 1)

    return body


def kernel(input, weight):
    NB, NS = input.shape
    V, D = weight.shape
    flat_idx = input.T.reshape(NB * NS)
    a5 = _gather_call(NB, NS, V, D)(flat_idx, weight)
    return a5.transpose(2, 4, 0, 1, 3).reshape(NB, NS, D)
